# baseline (speedup 1.0000x reference)
import functools

import jax
import jax.numpy as jnp
from jax import lax
from jax.experimental import pallas as pl
from jax.experimental.pallas import tpu as pltpu

_BLK = 8192
_INT_MAX = 2147483647


def _tc_kernel(data_ref, w_ref, b_ref, p_ref, lab_ref, data_any,
               row_ref, labout_ref, bv_s, bi_s, bl_s, row_v, sem, *,
               n_rows, nb):
    i = pl.program_id(0)
    x = data_ref[...]
    e = jnp.dot(x, w_ref[...], preferred_element_type=jnp.float32)
    diff = e + (b_ref[...] - p_ref[...])
    d2 = jnp.sum(diff * diff, axis=1, keepdims=True)
    riota = lax.broadcasted_iota(jnp.int32, (_BLK, 1), 0)
    d2 = jnp.where(riota < n_rows - i * _BLK, d2, jnp.inf)
    m = jnp.min(d2)

    @pl.when(i == 0)
    def _():
        bv_s[0] = jnp.float32(jnp.inf)
        bi_s[0] = jnp.int32(_INT_MAX)
        bl_s[0] = jnp.int32(0)

    better = m < bv_s[0]

    @pl.when(better)
    def _():
        d2r = d2.reshape(_BLK // 128, 128)
        ridx = (lax.broadcasted_iota(jnp.int32, d2r.shape, 0) * 128
                + lax.broadcasted_iota(jnp.int32, d2r.shape, 1))
        a_rel = jnp.min(jnp.where(d2r == m, ridx, _INT_MAX))
        lab_r = lab_ref[...].reshape(_BLK // 128, 128)
        bv_s[0] = m
        bi_s[0] = i * _BLK + a_rel
        bl_s[0] = jnp.max(jnp.where(ridx == a_rel, lab_r, 0))

    @pl.when(i == nb - 1)
    def _():
        idx = bi_s[0]
        rbase = (idx // 8) * 8
        pltpu.make_async_copy(
            data_any.at[pl.ds(rbase, 8)], row_v, sem).start()
        pltpu.make_async_copy(
            data_any.at[pl.ds(rbase, 8)], row_v, sem).wait()
        rows = row_v[...]
        rmask = lax.broadcasted_iota(jnp.int32, rows.shape, 0) == idx - rbase
        row_ref[...] = jnp.sum(jnp.where(rmask, rows, 0.0), axis=0,
                               keepdims=True)
        labout_ref[0, 0] = bl_s[0]


def kernel(prototype_vector, data, label, W, b):
    n, feat = data.shape
    latent = W.shape[1]
    nb = (n + _BLK - 1) // _BLK

    row, lab = pl.pallas_call(
        functools.partial(_tc_kernel, n_rows=n, nb=nb),
        grid=(nb,),
        in_specs=[
            pl.BlockSpec((_BLK, feat), lambda i: (i, 0)),
            pl.BlockSpec((feat, latent), lambda i: (0, 0)),
            pl.BlockSpec((1, latent), lambda i: (0, 0)),
            pl.BlockSpec((1, latent), lambda i: (0, 0)),
            pl.BlockSpec((_BLK, 1), lambda i: (i, 0)),
            pl.BlockSpec(memory_space=pltpu.MemorySpace.HBM),
        ],
        out_specs=[
            pl.BlockSpec((1, feat), lambda i: (0, 0)),
            pl.BlockSpec((1, 1), lambda i: (0, 0), memory_space=pltpu.SMEM),
        ],
        out_shape=[
            jax.ShapeDtypeStruct((1, feat), jnp.float32),
            jax.ShapeDtypeStruct((1, 1), jnp.int32),
        ],
        scratch_shapes=[
            pltpu.SMEM((1,), jnp.float32),
            pltpu.SMEM((1,), jnp.int32),
            pltpu.SMEM((1,), jnp.int32),
            pltpu.VMEM((8, feat), jnp.float32),
            pltpu.SemaphoreType.DMA,
        ],
    )(data, W, b.reshape(1, latent), prototype_vector.reshape(1, latent),
      label.reshape(n, 1), data)
    return (row.reshape(feat), lab[0, 0])


# speedup vs baseline: 2.4098x; 2.4098x over previous
import functools

import jax
import jax.numpy as jnp
from jax import lax
from jax.experimental import pallas as pl
from jax.experimental.pallas import tpu as pltpu

_BLK = 8192
_INT_MAX = 2147483647


def _tc_kernel(data_ref, w_ref, b_ref, p_ref, data_any, lab_any,
               row_ref, lab_ref, bv_s, bi_s, row_v, lab_s, sem, sem2, *,
               n_rows, nb):
    i = pl.program_id(0)
    x = data_ref[...]
    e = jnp.dot(x, w_ref[...], preferred_element_type=jnp.float32)
    diff = e + (b_ref[...] - p_ref[...])
    d2 = jnp.sum(diff * diff, axis=1, keepdims=True)
    d2r = d2.reshape(_BLK // 128, 128)
    ridx = (lax.broadcasted_iota(jnp.int32, d2r.shape, 0) * 128
            + lax.broadcasted_iota(jnp.int32, d2r.shape, 1))
    d2r = jnp.where((i * _BLK + ridx) < n_rows, d2r, jnp.inf)
    m = jnp.min(d2r)
    a = i * _BLK + jnp.min(jnp.where(d2r == m, ridx, _INT_MAX))

    @pl.when(i == 0)
    def _():
        bv_s[0] = jnp.float32(jnp.inf)
        bi_s[0] = jnp.int32(_INT_MAX)

    better = m < bv_s[0]
    bv_s[0] = jnp.where(better, m, bv_s[0])
    bi_s[0] = jnp.where(better, a, bi_s[0])

    @pl.when(i == nb - 1)
    def _():
        idx = bi_s[0]
        rbase = (idx // 8) * 8
        wbase = (idx // 128) * 128
        pltpu.make_async_copy(
            data_any.at[pl.ds(rbase, 8)], row_v, sem).start()
        pltpu.make_async_copy(
            lab_any.at[pl.ds(wbase, 128)], lab_s, sem2).start()
        pltpu.make_async_copy(
            data_any.at[pl.ds(rbase, 8)], row_v, sem).wait()
        pltpu.make_async_copy(
            lab_any.at[pl.ds(wbase, 128)], lab_s, sem2).wait()
        rows = row_v[...]
        rmask = lax.broadcasted_iota(jnp.int32, rows.shape, 0) == idx - rbase
        row_ref[...] = jnp.sum(jnp.where(rmask, rows, 0.0), axis=0,
                               keepdims=True)
        lab_ref[0, 0] = lab_s[idx - wbase]


def kernel(prototype_vector, data, label, W, b):
    n, feat = data.shape
    latent = W.shape[1]
    nb = (n + _BLK - 1) // _BLK

    row, lab = pl.pallas_call(
        functools.partial(_tc_kernel, n_rows=n, nb=nb),
        grid=(nb,),
        in_specs=[
            pl.BlockSpec((_BLK, feat), lambda i: (i, 0)),
            pl.BlockSpec((feat, latent), lambda i: (0, 0)),
            pl.BlockSpec((1, latent), lambda i: (0, 0)),
            pl.BlockSpec((1, latent), lambda i: (0, 0)),
            pl.BlockSpec(memory_space=pltpu.MemorySpace.HBM),
            pl.BlockSpec(memory_space=pltpu.MemorySpace.HBM),
        ],
        out_specs=[
            pl.BlockSpec((1, feat), lambda i: (0, 0)),
            pl.BlockSpec((1, 1), lambda i: (0, 0), memory_space=pltpu.SMEM),
        ],
        out_shape=[
            jax.ShapeDtypeStruct((1, feat), jnp.float32),
            jax.ShapeDtypeStruct((1, 1), jnp.int32),
        ],
        scratch_shapes=[
            pltpu.SMEM((1,), jnp.float32),
            pltpu.SMEM((1,), jnp.int32),
            pltpu.VMEM((8, feat), jnp.float32),
            pltpu.SMEM((128,), jnp.int32),
            pltpu.SemaphoreType.DMA,
            pltpu.SemaphoreType.DMA,
        ],
    )(data, W, b.reshape(1, latent), prototype_vector.reshape(1, latent),
      data, jnp.pad(label, (0, 128)))
    return (row.reshape(feat), lab[0, 0])
